# baseline (speedup 1.0000x reference)
import jax
import jax.numpy as jnp
from jax import lax
from jax.experimental import pallas as pl
from jax.experimental.pallas import tpu as pltpu
from jax.experimental.pallas import tpu_sc as plsc

D_MODEL = 64
SCALE = float(D_MODEL) ** 0.5

B_ROWS = 4096
B_COLS = 200
NRING = 4
NPAIR = B_COLS


def _sc_body(idx_hbm, table_hbm, out_hbm, idx_v,
             g0, g1, g2, g3, o0, o1, o2, o3,
             gs0, gs1, gs2, gs3, os0, os1, os2, os3):
    nc = 2
    wid = lax.axis_index("s") * nc + lax.axis_index("c")

    gbuf = (g0, g1, g2, g3)
    obuf = (o0, o1, o2, o3)
    gsem = (gs0, gs1, gs2, gs3)
    osem = (os0, os1, os2, os3)

    pltpu.sync_copy(idx_hbm.at[:, wid], idx_v)

    iota = lax.iota(jnp.int32, 16)

    def start_gather(p, n):
        ph = p // 8
        pr = p % 8
        pltpu.async_copy(table_hbm.at[idx_v.at[ph, pr]], gbuf[n], gsem[n])

    def out_slices(p):
        return out_hbm.at[p, :, wid]

    for n in range(NRING):
        start_gather(n, n)

    @pl.loop(0, NPAIR // NRING)
    def _outer(s):
        for n in range(NRING):
            p = s * NRING + n

            pltpu.make_async_copy(
                table_hbm.at[pl.ds(0, 128)], gbuf[n], gsem[n]).wait()

            @pl.when(p >= NRING)
            def _():
                pltpu.make_async_copy(
                    obuf[n], out_slices(p - NRING), osem[n]).wait()

            @plsc.parallel_loop(0, D_MODEL, unroll=2)
            def _d(d):
                dh = d // 8
                dl = d % 8
                col = jnp.full((16,), d, jnp.int32)
                for g in range(8):
                    rows = g * 16 + iota
                    vals = plsc.load_gather(gbuf[n], [rows, col])
                    obuf[n][dh, dl, pl.ds(g * 16, 16)] = vals * SCALE

            @pl.when(p + NRING < NPAIR)
            def _():
                start_gather(p + NRING, n)

            pltpu.async_copy(obuf[n], out_slices(p), osem[n])

    for n in range(NRING):
        p = NPAIR - NRING + n
        pltpu.make_async_copy(obuf[n], out_slices(p), osem[n]).wait()


@jax.jit
def _embed(x5, embedding):
    mesh = plsc.VectorSubcoreMesh(core_axis_name="c", subcore_axis_name="s")
    run = pl.kernel(
        _sc_body,
        out_type=jax.ShapeDtypeStruct((B_COLS, 8, 32, 8, 128), jnp.float32),
        mesh=mesh,
        compiler_params=pltpu.CompilerParams(
            use_tc_tiling_on_sc=False, needs_layout_passes=False),
        scratch_types=(
            [pltpu.VMEM((25, 8, 128), jnp.int32)]
            + [pltpu.VMEM((128, D_MODEL), jnp.float32)] * NRING
            + [pltpu.VMEM((8, 8, 128), jnp.float32)] * NRING
            + [pltpu.SemaphoreType.DMA] * (2 * NRING)
        ),
    )
    return run(x5, embedding)


def kernel(x, embedding):
    x5 = x.astype(jnp.int32).T.reshape(25, 8, 32, 128).transpose(0, 2, 1, 3)
    out5 = _embed(x5, embedding)
    return out5.transpose(2, 4, 0, 1, 3).reshape(B_ROWS, B_COLS, D_MODEL)


# speedup vs baseline: 1.6779x; 1.6779x over previous
import jax
import jax.numpy as jnp
from jax import lax
from jax.experimental import pallas as pl
from jax.experimental.pallas import tpu as pltpu
from jax.experimental.pallas import tpu_sc as plsc

D_MODEL = 64
SCALE = float(D_MODEL) ** 0.5

B_ROWS = 4096
B_COLS = 200
NRING = 4
NPAIR = B_COLS
OPAD = 137


def _sc_body(idx_hbm, table_hbm, out_hbm, idx_v,
             g0, g1, g2, g3, o0, o1, o2, o3,
             gs0, gs1, gs2, gs3, os0, os1, os2, os3):
    nc = 2
    wid = lax.axis_index("s") * nc + lax.axis_index("c")

    gbuf = (g0, g1, g2, g3)
    obuf = (o0, o1, o2, o3)
    gsem = (gs0, gs1, gs2, gs3)
    osem = (os0, os1, os2, os3)

    pltpu.sync_copy(idx_hbm.at[:, wid], idx_v)

    iota = lax.iota(jnp.int32, 16)

    dhv = [(k * 16 + iota) // 8 for k in range(4)]
    dlv = [(k * 16 + iota) % 8 for k in range(4)]

    def start_gather(p, n):
        ph = p // 8
        pr = p % 8
        pltpu.async_copy(table_hbm.at[idx_v.at[ph, pr]], gbuf[n], gsem[n])

    def out_slices(p):
        return out_hbm.at[p, :, wid]

    def obuf_view(n):
        return obuf[n].at[:, :, pl.ds(0, 128)]

    def wait_gather(n):
        pltpu.make_async_copy(
            table_hbm.at[pl.ds(0, 128)], gbuf[n], gsem[n]).wait()

    for n in range(NRING):
        start_gather(n, n)

    @pl.loop(0, NPAIR // NRING)
    def _outer(s):
        for n in range(NRING):
            p = s * NRING + n

            wait_gather(n)

            @pl.when(p >= NRING)
            def _():
                pltpu.make_async_copy(
                    obuf_view(n), out_slices(p - NRING), osem[n]).wait()

            @plsc.parallel_loop(0, 128)
            def _b(b):
                bcol = jnp.full((16,), b, jnp.int32)
                for k in range(4):
                    vals = gbuf[n][b, pl.ds(k * 16, 16)]
                    plsc.store_scatter(obuf[n], [dhv[k], dlv[k], bcol],
                                       vals * SCALE)

            @pl.when(p + NRING < NPAIR)
            def _():
                start_gather(p + NRING, n)

            pltpu.async_copy(obuf_view(n), out_slices(p), osem[n])

    for n in range(NRING):
        p = NPAIR - NRING + n
        pltpu.make_async_copy(obuf_view(n), out_slices(p), osem[n]).wait()


@jax.jit
def _embed(x5, embedding):
    mesh = plsc.VectorSubcoreMesh(core_axis_name="c", subcore_axis_name="s")
    run = pl.kernel(
        _sc_body,
        out_type=jax.ShapeDtypeStruct((B_COLS, 8, 32, 8, 128), jnp.float32),
        mesh=mesh,
        compiler_params=pltpu.CompilerParams(
            use_tc_tiling_on_sc=False, needs_layout_passes=False),
        scratch_types=(
            [pltpu.VMEM((25, 8, 128), jnp.int32)]
            + [pltpu.VMEM((128, D_MODEL), jnp.float32)] * NRING
            + [pltpu.VMEM((8, 8, OPAD), jnp.float32)] * NRING
            + [pltpu.SemaphoreType.DMA] * (2 * NRING)
        ),
    )
    return run(x5, embedding)


def kernel(x, embedding):
    x5 = x.astype(jnp.int32).T.reshape(25, 8, 32, 128).transpose(0, 2, 1, 3)
    out5 = _embed(x5, embedding)
    return out5.transpose(2, 4, 0, 1, 3).reshape(B_ROWS, B_COLS, D_MODEL)
